# Initial kernel scaffold; baseline (speedup 1.0000x reference)
import jax
import jax.numpy as jnp
from jax.experimental import pallas as pl


def kernel(taxid, genus, family, device_type, country, duration, emb_taxid, emb_genus, emb_family, Wt, bt, Wc, bc, Wn, bn):
    raise NotImplementedError("write your pallas kernel here")


# speedup vs baseline: 1.5201x; 1.5201x over previous
import functools

import jax
import jax.numpy as jnp
from jax import lax
from jax.experimental import pallas as pl
from jax.experimental.pallas import tpu as pltpu
from jax.experimental.pallas import tpu_sc as plsc

NB = 16384
NC, NS = 2, 16
NW = NC * NS
BPW = NB // NW
CH = 128
NCH = BPW // CH

DT, DG, DF = 64, 32, 16
BS = 2048


def _sc_gather_body(tax_hbm, gen_hbm, fam_hbm, et_hbm, eg_hbm, ef_hbm,
                    t_out, g_out, f_out,
                    idx_t, idx_g, idx_f, rows_t, rows_g, rows_f, sem):
    wid = lax.axis_index("s") * NC + lax.axis_index("c")
    base = wid * BPW
    crow = wid * NCH
    pltpu.sync_copy(tax_hbm.at[pl.ds(crow, NCH)], idx_t)
    pltpu.sync_copy(gen_hbm.at[pl.ds(crow, NCH)], idx_g)
    pltpu.sync_copy(fam_hbm.at[pl.ds(crow, NCH)], idx_f)
    copies = []
    for j in range(NCH):
        copies.append(pltpu.async_copy(
            et_hbm.at[idx_t.at[j]], rows_t.at[pl.ds(j * CH, CH)], sem))
        copies.append(pltpu.async_copy(
            eg_hbm.at[idx_g.at[j]], rows_g.at[pl.ds(j * CH, CH)], sem))
        copies.append(pltpu.async_copy(
            ef_hbm.at[idx_f.at[j]], rows_f.at[pl.ds(j * CH, CH)], sem))
    for c in copies:
        c.wait()
    pltpu.sync_copy(rows_t, t_out.at[pl.ds(base, BPW)])
    pltpu.sync_copy(rows_g, g_out.at[pl.ds(base, BPW)])
    pltpu.sync_copy(rows_f, f_out.at[pl.ds(base, BPW)])


@functools.lru_cache(maxsize=1)
def _make_sc_gather():
    return pl.kernel(
        _sc_gather_body,
        mesh=plsc.VectorSubcoreMesh(core_axis_name="c", subcore_axis_name="s"),
        compiler_params=pltpu.CompilerParams(use_tc_tiling_on_sc=False),
        out_type=[
            jax.ShapeDtypeStruct((NB, DT), jnp.float32),
            jax.ShapeDtypeStruct((NB, DG), jnp.float32),
            jax.ShapeDtypeStruct((NB, DF), jnp.float32),
        ],
        scratch_types=[
            pltpu.VMEM((NCH, CH), jnp.int32),
            pltpu.VMEM((NCH, CH), jnp.int32),
            pltpu.VMEM((NCH, CH), jnp.int32),
            pltpu.VMEM((BPW, DT), jnp.float32),
            pltpu.VMEM((BPW, DG), jnp.float32),
            pltpu.VMEM((BPW, DF), jnp.float32),
            pltpu.SemaphoreType.DMA,
        ],
    )


def _tc_body(t_ref, g_ref, f_ref, dt_ref, cty_ref, dur_ref,
             wt0_ref, wt1_ref, wt2_ref, bt_ref, wcd_ref, wcc_ref,
             bc_ref, wn_ref, bn_ref, out_ref):
    tax = (jnp.dot(t_ref[...], wt0_ref[...], preferred_element_type=jnp.float32)
           + jnp.dot(g_ref[...], wt1_ref[...], preferred_element_type=jnp.float32)
           + jnp.dot(f_ref[...], wt2_ref[...], preferred_element_type=jnp.float32)
           + bt_ref[...])
    tax = jnp.maximum(tax, 0.0)
    oh_d = (lax.broadcasted_iota(jnp.int32, (BS, 16), 1)
            == dt_ref[...]).astype(jnp.float32)
    oh_c = (lax.broadcasted_iota(jnp.int32, (BS, 256), 1)
            == cty_ref[...]).astype(jnp.float32)
    cat = (jnp.dot(oh_d, wcd_ref[...], preferred_element_type=jnp.float32)
           + jnp.dot(oh_c, wcc_ref[...], preferred_element_type=jnp.float32)
           + bc_ref[...])
    cat = jnp.maximum(cat, 0.0)
    num = jnp.maximum(dur_ref[...] * wn_ref[...] + bn_ref[...], 0.0)
    out_ref[...] = jnp.concatenate([tax, cat, num], axis=-1)


def kernel(taxid, genus, family, device_type, country, duration,
           emb_taxid, emb_genus, emb_family, Wt, bt, Wc, bc, Wn, bn):
    tax2 = taxid.astype(jnp.int32).reshape(NB // CH, CH)
    gen2 = genus.astype(jnp.int32).reshape(NB // CH, CH)
    fam2 = family.astype(jnp.int32).reshape(NB // CH, CH)
    t, g, f = _make_sc_gather()(tax2, gen2, fam2, emb_taxid, emb_genus, emb_family)

    dt2 = device_type.astype(jnp.int32).reshape(NB, 1)
    cty2 = country.astype(jnp.int32).reshape(NB, 1)
    dur2 = duration.reshape(NB, 1)
    wt0, wt1, wt2 = Wt[0:DT], Wt[DT:DT + DG], Wt[DT + DG:]
    bt2 = bt.reshape(1, DT)
    wcd, wcc = Wc[0:16], Wc[16:]
    bc2 = bc.reshape(1, 16)
    bn2 = bn.reshape(1, 16)

    out = pl.pallas_call(
        _tc_body,
        grid=(NB // BS,),
        in_specs=[
            pl.BlockSpec((BS, DT), lambda i: (i, 0)),
            pl.BlockSpec((BS, DG), lambda i: (i, 0)),
            pl.BlockSpec((BS, DF), lambda i: (i, 0)),
            pl.BlockSpec((BS, 1), lambda i: (i, 0)),
            pl.BlockSpec((BS, 1), lambda i: (i, 0)),
            pl.BlockSpec((BS, 1), lambda i: (i, 0)),
            pl.BlockSpec((DT, 64), lambda i: (0, 0)),
            pl.BlockSpec((DG, 64), lambda i: (0, 0)),
            pl.BlockSpec((DF, 64), lambda i: (0, 0)),
            pl.BlockSpec((1, DT), lambda i: (0, 0)),
            pl.BlockSpec((16, 16), lambda i: (0, 0)),
            pl.BlockSpec((256, 16), lambda i: (0, 0)),
            pl.BlockSpec((1, 16), lambda i: (0, 0)),
            pl.BlockSpec((1, 16), lambda i: (0, 0)),
            pl.BlockSpec((1, 16), lambda i: (0, 0)),
        ],
        out_specs=pl.BlockSpec((BS, 96), lambda i: (i, 0)),
        out_shape=jax.ShapeDtypeStruct((NB, 96), jnp.float32),
    )(t, g, f, dt2, cty2, dur2, wt0, wt1, wt2, bt2, wcd, wcc, bc2, Wn, bn2)
    return out


# speedup vs baseline: 1.6264x; 1.0699x over previous
import functools

import jax
import jax.numpy as jnp
from jax import lax
from jax.experimental import pallas as pl
from jax.experimental.pallas import tpu as pltpu
from jax.experimental.pallas import tpu_sc as plsc

NB = 16384
NC, NS = 2, 16
NW = NC * NS
BPW = NB // NW
CH = 128
NCH = BPW // CH
NJOB = 3 * NCH
NBUF = 4

VT, VG, VF = 100000, 10000, 2000
DT, DG, DF = 64, 32, 16
BS = 2048


def _proj_body(tv_ref, w_ref, out_ref):
    proj = lax.dot_general(tv_ref[...], w_ref[...],
                           (((0,), (0,)), ((), ())),
                           preferred_element_type=jnp.float32)
    out_ref[...] = jnp.concatenate(
        [proj, jnp.zeros_like(proj)], axis=-1)


def _project(table_t, w, c_blk):
    d, v = table_t.shape
    return pl.pallas_call(
        _proj_body,
        grid=(pl.cdiv(v, c_blk),),
        in_specs=[
            pl.BlockSpec((d, c_blk), lambda i: (0, i)),
            pl.BlockSpec((d, 64), lambda i: (0, 0)),
        ],
        out_specs=pl.BlockSpec((c_blk, 128), lambda i: (i, 0)),
        out_shape=jax.ShapeDtypeStruct((v, 128), jnp.float32),
    )(table_t, w)


def _sc_gather_body(tax_hbm, gen_hbm, fam_hbm, pt_hbm, pg_hbm, pf_hbm,
                    t_out, g_out, f_out,
                    idx_t, idx_g, idx_f, bufs, gsem, wsem):
    wid = lax.axis_index("s") * NC + lax.axis_index("c")
    base = wid * BPW
    crow = wid * NCH
    pltpu.sync_copy(tax_hbm.at[pl.ds(crow, NCH)], idx_t)
    pltpu.sync_copy(gen_hbm.at[pl.ds(crow, NCH)], idx_g)
    pltpu.sync_copy(fam_hbm.at[pl.ds(crow, NCH)], idx_f)
    jobs = []
    for tbl, idx in ((pt_hbm, idx_t), (pg_hbm, idx_g), (pf_hbm, idx_f)):
        for j in range(NCH):
            jobs.append((tbl, idx, j))
    outs = (t_out, g_out, f_out)

    def fire(k):
        tbl, idx, j = jobs[k]
        return pltpu.async_copy(tbl.at[idx.at[j]], bufs.at[k % NBUF], gsem)

    gath = [None] * NJOB
    wb = [None] * NJOB
    gath[0] = fire(0)
    for k in range(NJOB):
        if k + 1 < NJOB:
            if k + 1 >= NBUF:
                wb[k + 1 - NBUF].wait()
            gath[k + 1] = fire(k + 1)
        gath[k].wait()
        _, _, j = jobs[k]
        wb[k] = pltpu.async_copy(
            bufs.at[k % NBUF],
            outs[k // NCH].at[pl.ds(base + j * CH, CH)], wsem)
    for k in range(NJOB - NBUF, NJOB):
        wb[k].wait()


@functools.lru_cache(maxsize=1)
def _make_sc_gather():
    return pl.kernel(
        _sc_gather_body,
        mesh=plsc.VectorSubcoreMesh(core_axis_name="c", subcore_axis_name="s"),
        out_type=[
            jax.ShapeDtypeStruct((NB, 128), jnp.float32),
            jax.ShapeDtypeStruct((NB, 128), jnp.float32),
            jax.ShapeDtypeStruct((NB, 128), jnp.float32),
        ],
        scratch_types=[
            pltpu.VMEM((NCH, CH), jnp.int32),
            pltpu.VMEM((NCH, CH), jnp.int32),
            pltpu.VMEM((NCH, CH), jnp.int32),
            pltpu.VMEM((NBUF, CH, 128), jnp.float32),
            pltpu.SemaphoreType.DMA,
            pltpu.SemaphoreType.DMA,
        ],
    )


def _tc_body(t_ref, g_ref, f_ref, dt_ref, cty_ref, dur_ref,
             bt_ref, wcd_ref, wcc_ref, bc_ref, wn_ref, bn_ref, out_ref):
    tax = (t_ref[:, :64] + g_ref[:, :64] + f_ref[:, :64] + bt_ref[...])
    tax = jnp.maximum(tax, 0.0)
    oh_d = (lax.broadcasted_iota(jnp.int32, (BS, 16), 1)
            == dt_ref[...]).astype(jnp.float32)
    oh_c = (lax.broadcasted_iota(jnp.int32, (BS, 256), 1)
            == cty_ref[...]).astype(jnp.float32)
    cat = (jnp.dot(oh_d, wcd_ref[...], preferred_element_type=jnp.float32)
           + jnp.dot(oh_c, wcc_ref[...], preferred_element_type=jnp.float32)
           + bc_ref[...])
    cat = jnp.maximum(cat, 0.0)
    num = jnp.maximum(dur_ref[...] * wn_ref[...] + bn_ref[...], 0.0)
    out_ref[...] = jnp.concatenate([tax, cat, num], axis=-1)


def kernel(taxid, genus, family, device_type, country, duration,
           emb_taxid, emb_genus, emb_family, Wt, bt, Wc, bc, Wn, bn):
    pt = _project(emb_taxid.T, Wt[0:DT], 2048)
    pg = _project(emb_genus.T, Wt[DT:DT + DG], 2048)
    pf = _project(emb_family.T, Wt[DT + DG:], 2048)

    tax2 = taxid.astype(jnp.int32).reshape(NB // CH, CH)
    gen2 = genus.astype(jnp.int32).reshape(NB // CH, CH)
    fam2 = family.astype(jnp.int32).reshape(NB // CH, CH)
    t, g, f = _make_sc_gather()(tax2, gen2, fam2, pt, pg, pf)

    dt2 = device_type.astype(jnp.int32).reshape(NB, 1)
    cty2 = country.astype(jnp.int32).reshape(NB, 1)
    dur2 = duration.reshape(NB, 1)
    bt2 = bt.reshape(1, DT)
    wcd, wcc = Wc[0:16], Wc[16:]
    bc2 = bc.reshape(1, 16)
    bn2 = bn.reshape(1, 16)

    out = pl.pallas_call(
        _tc_body,
        grid=(NB // BS,),
        in_specs=[
            pl.BlockSpec((BS, 128), lambda i: (i, 0)),
            pl.BlockSpec((BS, 128), lambda i: (i, 0)),
            pl.BlockSpec((BS, 128), lambda i: (i, 0)),
            pl.BlockSpec((BS, 1), lambda i: (i, 0)),
            pl.BlockSpec((BS, 1), lambda i: (i, 0)),
            pl.BlockSpec((BS, 1), lambda i: (i, 0)),
            pl.BlockSpec((1, DT), lambda i: (0, 0)),
            pl.BlockSpec((16, 16), lambda i: (0, 0)),
            pl.BlockSpec((256, 16), lambda i: (0, 0)),
            pl.BlockSpec((1, 16), lambda i: (0, 0)),
            pl.BlockSpec((1, 16), lambda i: (0, 0)),
            pl.BlockSpec((1, 16), lambda i: (0, 0)),
        ],
        out_specs=pl.BlockSpec((BS, 96), lambda i: (i, 0)),
        out_shape=jax.ShapeDtypeStruct((NB, 96), jnp.float32),
    )(t, g, f, dt2, cty2, dur2, bt2, wcd, wcc, bc2, Wn, bn2)
    return out


# speedup vs baseline: 2.1946x; 1.3493x over previous
import functools

import jax
import jax.numpy as jnp
from jax import lax
from jax.experimental import pallas as pl
from jax.experimental.pallas import tpu as pltpu
from jax.experimental.pallas import tpu_sc as plsc

NB = 16384
NC, NS = 2, 16
NW = NC * NS
BPW = NB // NW
CH = 128
NCH = BPW // CH
NJOB = 3 * NCH
NBUF = 6

DT, DG, DF = 64, 32, 16
PB = 2048
BS = 2048


def _proj2_body(tva_ref, tvb_ref, w_ref, out_ref):
    pa = lax.dot_general(tva_ref[...], w_ref[...],
                         (((0,), (0,)), ((), ())),
                         preferred_element_type=jnp.float32)
    pb = lax.dot_general(tvb_ref[...], w_ref[...],
                         (((0,), (0,)), ((), ())),
                         preferred_element_type=jnp.float32)
    out_ref[...] = jnp.concatenate([pa, pb], axis=-1)


def _project2(table_t, w):
    d, v = table_t.shape
    nblk = pl.cdiv(v, PB)
    npair = pl.cdiv(nblk, 2)
    last = nblk - 1
    return pl.pallas_call(
        _proj2_body,
        grid=(npair,),
        in_specs=[
            pl.BlockSpec((d, PB), lambda i: (0, 2 * i)),
            pl.BlockSpec((d, PB), lambda i: (0, jnp.minimum(2 * i + 1, last))),
            pl.BlockSpec((d, 64), lambda i: (0, 0)),
        ],
        out_specs=pl.BlockSpec((PB, 128), lambda i: (i, 0)),
        out_shape=jax.ShapeDtypeStruct((npair * PB, 128), jnp.float32),
    )(table_t, table_t, w)


def _sc_gather_body(tax_hbm, gen_hbm, fam_hbm, pt_hbm, pg_hbm, pf_hbm,
                    t_out, g_out, f_out,
                    idx_t, idx_g, idx_f, bufs, gsem, wsem):
    wid = lax.axis_index("s") * NC + lax.axis_index("c")
    base = wid * BPW
    crow = wid * NCH
    pltpu.sync_copy(tax_hbm.at[pl.ds(crow, NCH)], idx_t)
    pltpu.sync_copy(gen_hbm.at[pl.ds(crow, NCH)], idx_g)
    pltpu.sync_copy(fam_hbm.at[pl.ds(crow, NCH)], idx_f)
    jobs = []
    for tbl, idx in ((pt_hbm, idx_t), (pg_hbm, idx_g), (pf_hbm, idx_f)):
        for j in range(NCH):
            jobs.append((tbl, idx, j))
    outs = (t_out, g_out, f_out)

    def fire(k):
        tbl, idx, j = jobs[k]
        return pltpu.async_copy(tbl.at[idx.at[j]], bufs.at[k % NBUF], gsem)

    gath = [None] * NJOB
    wb = [None] * NJOB
    gath[0] = fire(0)
    for k in range(NJOB):
        if k + 1 < NJOB:
            if k + 1 >= NBUF:
                wb[k + 1 - NBUF].wait()
            gath[k + 1] = fire(k + 1)
        gath[k].wait()
        _, _, j = jobs[k]
        wb[k] = pltpu.async_copy(
            bufs.at[k % NBUF],
            outs[k // NCH].at[pl.ds(base + j * CH, CH)], wsem)
    for k in range(NJOB - NBUF, NJOB):
        wb[k].wait()


@functools.lru_cache(maxsize=1)
def _make_sc_gather(vt_pack, vg_pack, vf_pack):
    return pl.kernel(
        _sc_gather_body,
        mesh=plsc.VectorSubcoreMesh(core_axis_name="c", subcore_axis_name="s"),
        out_type=[
            jax.ShapeDtypeStruct((NB, 128), jnp.float32),
            jax.ShapeDtypeStruct((NB, 128), jnp.float32),
            jax.ShapeDtypeStruct((NB, 128), jnp.float32),
        ],
        scratch_types=[
            pltpu.VMEM((NCH, CH), jnp.int32),
            pltpu.VMEM((NCH, CH), jnp.int32),
            pltpu.VMEM((NCH, CH), jnp.int32),
            pltpu.VMEM((NBUF, CH, 128), jnp.float32),
            pltpu.SemaphoreType.DMA,
            pltpu.SemaphoreType.DMA,
        ],
    )


def _half(x_t, sel):
    return jnp.where(sel, x_t[64:, :], x_t[:64, :])


def _tc_body(t_ref, g_ref, f_ref, sel_ref, dt_ref, cty_ref, dur_ref,
             bt_ref, wcd_ref, wcc_ref, bc_ref, wn_ref, bn_ref, out_ref):
    sbits = sel_ref[...][None, :]
    t_t = jnp.transpose(t_ref[...])
    g_t = jnp.transpose(g_ref[...])
    f_t = jnp.transpose(f_ref[...])
    tax = (_half(t_t, (sbits & 1) > 0)
           + _half(g_t, (sbits & 2) > 0)
           + _half(f_t, (sbits & 4) > 0)
           + bt_ref[...])
    tax = jnp.maximum(tax, 0.0)
    dt_row = dt_ref[...][None, :]
    cty_row = cty_ref[...][None, :]
    oh_d = (lax.broadcasted_iota(jnp.int32, (16, BS), 0)
            == dt_row).astype(jnp.float32)
    oh_c = (lax.broadcasted_iota(jnp.int32, (256, BS), 0)
            == cty_row).astype(jnp.float32)
    cat = (lax.dot_general(wcd_ref[...], oh_d, (((0,), (0,)), ((), ())),
                           preferred_element_type=jnp.float32)
           + lax.dot_general(wcc_ref[...], oh_c, (((0,), (0,)), ((), ())),
                             preferred_element_type=jnp.float32)
           + bc_ref[...])
    cat = jnp.maximum(cat, 0.0)
    dur_row = dur_ref[...][None, :]
    num = jnp.maximum(
        lax.dot_general(wn_ref[...], dur_row, (((0,), (0,)), ((), ())),
                        preferred_element_type=jnp.float32) + bn_ref[...],
        0.0)
    out_ref[...] = jnp.concatenate([tax, cat, num], axis=0)


def _pack_idx(idx, v):
    idx = idx.astype(jnp.int32)
    row = (idx >> 12) * PB + (idx & (PB - 1))
    side = (idx >> 11) & 1
    return row, side


def kernel(taxid, genus, family, device_type, country, duration,
           emb_taxid, emb_genus, emb_family, Wt, bt, Wc, bc, Wn, bn):
    pt = _project2(emb_taxid.T, Wt[0:DT])
    pg = _project2(emb_genus.T, Wt[DT:DT + DG])
    pf = _project2(emb_family.T, Wt[DT + DG:])

    row_t, side_t = _pack_idx(taxid, pt.shape[0])
    row_g, side_g = _pack_idx(genus, pg.shape[0])
    row_f, side_f = _pack_idx(family, pf.shape[0])
    selbits = (side_t | (side_g << 1) | (side_f << 2)).astype(jnp.int32)

    t, g, f = _make_sc_gather(pt.shape[0], pg.shape[0], pf.shape[0])(
        row_t.reshape(NB // CH, CH), row_g.reshape(NB // CH, CH),
        row_f.reshape(NB // CH, CH), pt, pg, pf)

    dt1 = device_type.astype(jnp.int32)
    cty1 = country.astype(jnp.int32)
    bt2 = bt.reshape(DT, 1)
    wcd, wcc = Wc[0:16], Wc[16:]
    bc2 = bc.reshape(16, 1)
    bn2 = bn.reshape(16, 1)

    out_t = pl.pallas_call(
        _tc_body,
        grid=(NB // BS,),
        in_specs=[
            pl.BlockSpec((BS, 128), lambda i: (i, 0)),
            pl.BlockSpec((BS, 128), lambda i: (i, 0)),
            pl.BlockSpec((BS, 128), lambda i: (i, 0)),
            pl.BlockSpec((BS,), lambda i: (i,)),
            pl.BlockSpec((BS,), lambda i: (i,)),
            pl.BlockSpec((BS,), lambda i: (i,)),
            pl.BlockSpec((BS,), lambda i: (i,)),
            pl.BlockSpec((DT, 1), lambda i: (0, 0)),
            pl.BlockSpec((16, 16), lambda i: (0, 0)),
            pl.BlockSpec((256, 16), lambda i: (0, 0)),
            pl.BlockSpec((16, 1), lambda i: (0, 0)),
            pl.BlockSpec((1, 16), lambda i: (0, 0)),
            pl.BlockSpec((16, 1), lambda i: (0, 0)),
        ],
        out_specs=pl.BlockSpec((96, BS), lambda i: (0, i)),
        out_shape=jax.ShapeDtypeStruct((96, NB), jnp.float32),
    )(t, g, f, selbits, dt1, cty1, duration,
      bt2, wcd, wcc, bc2, Wn, bn2)
    return out_t.T


# speedup vs baseline: 2.2742x; 1.0363x over previous
import functools

import jax
import jax.numpy as jnp
from jax import lax
from jax.experimental import pallas as pl
from jax.experimental.pallas import tpu as pltpu
from jax.experimental.pallas import tpu_sc as plsc

NB = 16384
NC, NS = 2, 16
NW = NC * NS
BPW = NB // NW
CH = 128
NCH = BPW // CH
NJOB = 3 * NCH
NBUF = 6

DT, DG, DF = 64, 32, 16
PB = 2048
BS = 2048


def _proj2_body(tva_ref, tvb_ref, w_ref, out_ref):
    pa = lax.dot_general(tva_ref[...], w_ref[...],
                         (((0,), (0,)), ((), ())),
                         precision=lax.Precision.DEFAULT,
                         preferred_element_type=jnp.float32)
    pb = lax.dot_general(tvb_ref[...], w_ref[...],
                         (((0,), (0,)), ((), ())),
                         precision=lax.Precision.DEFAULT,
                         preferred_element_type=jnp.float32)
    out_ref[...] = jnp.concatenate([pa, pb], axis=-1)


def _project2(table_t, w):
    d, v = table_t.shape
    nblk = pl.cdiv(v, PB)
    npair = pl.cdiv(nblk, 2)
    last = nblk - 1
    return pl.pallas_call(
        _proj2_body,
        grid=(npair,),
        in_specs=[
            pl.BlockSpec((d, PB), lambda i: (0, 2 * i)),
            pl.BlockSpec((d, PB), lambda i: (0, jnp.minimum(2 * i + 1, last))),
            pl.BlockSpec((d, 64), lambda i: (0, 0)),
        ],
        out_specs=pl.BlockSpec((PB, 128), lambda i: (i, 0)),
        out_shape=jax.ShapeDtypeStruct((npair * PB, 128), jnp.float32),
    )(table_t, table_t, w)


def _sc_body_factory(ntab):
    njob = ntab * NCH
    nbuf = min(NBUF, njob)

    def body(*refs):
        idx_hbms = refs[0:ntab]
        tbls = refs[ntab:2 * ntab]
        outs = refs[2 * ntab:3 * ntab]
        idxs = refs[3 * ntab:4 * ntab]
        bufs = refs[4 * ntab]
        gsem = refs[4 * ntab + 1]
        wsem = refs[4 * ntab + 2]
        wid = lax.axis_index("s") * NC + lax.axis_index("c")
        base = wid * BPW
        crow = wid * NCH
        for ih, iv in zip(idx_hbms, idxs):
            pltpu.sync_copy(ih.at[pl.ds(crow, NCH)], iv)
        jobs = [(tbls[tt], idxs[tt], j)
                for tt in range(ntab) for j in range(NCH)]

        def fire(k):
            tbl, idx, j = jobs[k]
            return pltpu.async_copy(tbl.at[idx.at[j]], bufs.at[k % nbuf], gsem)

        gath = [None] * njob
        wb = [None] * njob
        gath[0] = fire(0)
        for k in range(njob):
            if k + 1 < njob:
                if k + 1 >= nbuf:
                    wb[k + 1 - nbuf].wait()
                gath[k + 1] = fire(k + 1)
            gath[k].wait()
            _, _, j = jobs[k]
            wb[k] = pltpu.async_copy(
                bufs.at[k % nbuf],
                outs[k // NCH].at[pl.ds(base + j * CH, CH)], wsem)
        for k in range(njob - nbuf, njob):
            wb[k].wait()

    return body


@functools.lru_cache(maxsize=4)
def _make_sc_gather(ntab):
    nbuf = min(NBUF, ntab * NCH)
    return pl.kernel(
        _sc_body_factory(ntab),
        mesh=plsc.VectorSubcoreMesh(core_axis_name="c", subcore_axis_name="s"),
        out_type=[jax.ShapeDtypeStruct((NB, 128), jnp.float32)
                  for _ in range(ntab)],
        scratch_types=(
            [pltpu.VMEM((NCH, CH), jnp.int32) for _ in range(ntab)]
            + [pltpu.VMEM((nbuf, CH, 128), jnp.float32),
               pltpu.SemaphoreType.DMA,
               pltpu.SemaphoreType.DMA]
        ),
    )


def _half(x_t, sel):
    return jnp.where(sel, x_t[64:, :], x_t[:64, :])


def _tc_body(t_ref, g_ref, f_ref, sel_ref, dt_ref, cty_ref, dur_ref,
             bt_ref, wcd_ref, wcc_ref, bc_ref, wn_ref, bn_ref, out_ref):
    sbits = sel_ref[...][None, :]
    t_t = jnp.transpose(t_ref[...])
    g_t = jnp.transpose(g_ref[...])
    f_t = jnp.transpose(f_ref[...])
    tax = (_half(t_t, (sbits & 1) > 0)
           + _half(g_t, (sbits & 2) > 0)
           + _half(f_t, (sbits & 4) > 0)
           + bt_ref[...])
    tax = jnp.maximum(tax, 0.0)
    dt_row = dt_ref[...][None, :]
    cty_row = cty_ref[...][None, :]
    oh_d = (lax.broadcasted_iota(jnp.int32, (16, BS), 0)
            == dt_row).astype(jnp.float32)
    oh_c = (lax.broadcasted_iota(jnp.int32, (256, BS), 0)
            == cty_row).astype(jnp.float32)
    cat = (lax.dot_general(wcd_ref[...], oh_d, (((0,), (0,)), ((), ())),
                           preferred_element_type=jnp.float32)
           + lax.dot_general(wcc_ref[...], oh_c, (((0,), (0,)), ((), ())),
                             preferred_element_type=jnp.float32)
           + bc_ref[...])
    cat = jnp.maximum(cat, 0.0)
    dur_row = dur_ref[...][None, :]
    num = jnp.maximum(
        lax.dot_general(wn_ref[...], dur_row, (((0,), (0,)), ((), ())),
                        preferred_element_type=jnp.float32) + bn_ref[...],
        0.0)
    out_ref[...] = jnp.concatenate([tax, cat, num], axis=0)


def _pack_idx(idx, v):
    idx = idx.astype(jnp.int32)
    row = (idx >> 12) * PB + (idx & (PB - 1))
    side = (idx >> 11) & 1
    return row, side


def kernel(taxid, genus, family, device_type, country, duration,
           emb_taxid, emb_genus, emb_family, Wt, bt, Wc, bc, Wn, bn):
    pg = _project2(emb_genus.T, Wt[DT:DT + DG])
    pf = _project2(emb_family.T, Wt[DT + DG:])

    row_g, side_g = _pack_idx(genus, pg.shape[0])
    row_f, side_f = _pack_idx(family, pf.shape[0])
    g, f = _make_sc_gather(2)(
        row_g.reshape(NB // CH, CH), row_f.reshape(NB // CH, CH), pg, pf)

    pt = _project2(emb_taxid.T, Wt[0:DT])
    row_t, side_t = _pack_idx(taxid, pt.shape[0])
    selbits = (side_t | (side_g << 1) | (side_f << 2)).astype(jnp.int32)
    t = _make_sc_gather(1)(row_t.reshape(NB // CH, CH), pt)
    if isinstance(t, (list, tuple)):
        t = t[0]

    dt1 = device_type.astype(jnp.int32)
    cty1 = country.astype(jnp.int32)
    bt2 = bt.reshape(DT, 1)
    wcd, wcc = Wc[0:16], Wc[16:]
    bc2 = bc.reshape(16, 1)
    bn2 = bn.reshape(16, 1)

    out_t = pl.pallas_call(
        _tc_body,
        grid=(NB // BS,),
        in_specs=[
            pl.BlockSpec((BS, 128), lambda i: (i, 0)),
            pl.BlockSpec((BS, 128), lambda i: (i, 0)),
            pl.BlockSpec((BS, 128), lambda i: (i, 0)),
            pl.BlockSpec((BS,), lambda i: (i,)),
            pl.BlockSpec((BS,), lambda i: (i,)),
            pl.BlockSpec((BS,), lambda i: (i,)),
            pl.BlockSpec((BS,), lambda i: (i,)),
            pl.BlockSpec((DT, 1), lambda i: (0, 0)),
            pl.BlockSpec((16, 16), lambda i: (0, 0)),
            pl.BlockSpec((256, 16), lambda i: (0, 0)),
            pl.BlockSpec((16, 1), lambda i: (0, 0)),
            pl.BlockSpec((1, 16), lambda i: (0, 0)),
            pl.BlockSpec((16, 1), lambda i: (0, 0)),
        ],
        out_specs=pl.BlockSpec((96, BS), lambda i: (0, i)),
        out_shape=jax.ShapeDtypeStruct((96, NB), jnp.float32),
    )(t, g, f, selbits, dt1, cty1, duration,
      bt2, wcd, wcc, bc2, Wn, bn2)
    return out_t.T
